# Initial kernel scaffold; baseline (speedup 1.0000x reference)
import jax
import jax.numpy as jnp
from jax.experimental import pallas as pl


def kernel(batch_pos_list, table):
    raise NotImplementedError("write your pallas kernel here")


# speedup vs baseline: 3.7084x; 3.7084x over previous
import functools

import jax
import jax.numpy as jnp
from jax import lax
from jax.experimental import pallas as pl
from jax.experimental.pallas import tpu as pltpu
from jax.experimental.pallas import tpu_sc as plsc

DIM = 64
LANES = 16
IDX_ROW = 128
T_CHUNK = 512


def _make_kernel(num_tokens, vocab):
    info = plsc.get_sparse_core_info()
    num_workers = info.num_cores * info.num_subcores
    per_w = num_tokens // num_workers
    assert per_w * num_workers == num_tokens
    assert per_w % T_CHUNK == 0
    n_chunks = per_w // T_CHUNK
    n_streams = (2 * T_CHUNK) // IDX_ROW
    idx_rows_per_chunk = (2 * T_CHUNK) // IDX_ROW

    mesh = plsc.VectorSubcoreMesh(core_axis_name="c", subcore_axis_name="s")

    @functools.partial(
        pl.kernel,
        mesh=mesh,
        out_type=jax.ShapeDtypeStruct((num_tokens, DIM), jnp.float32),
        scratch_types=[
            pltpu.VMEM((idx_rows_per_chunk, IDX_ROW), jnp.int32),
            pltpu.VMEM((2 * T_CHUNK, DIM), jnp.float32),
            pltpu.VMEM((T_CHUNK, DIM), jnp.float32),
            pltpu.SemaphoreType.DMA,
        ],
        compiler_params=pltpu.CompilerParams(use_tc_tiling_on_sc=False),
    )
    def k(idx_hbm, table_hbm, out_hbm, idx_v, rows_v, out_v, sem):
        wid = lax.axis_index("s") * info.num_cores + lax.axis_index("c")
        tok0 = wid * per_w

        def chunk_body(g, carry):
            base = pl.multiple_of(tok0 + g * T_CHUNK, T_CHUNK)
            idx_row0 = pl.multiple_of((base * 2) // IDX_ROW, 8)
            pltpu.sync_copy(idx_hbm.at[pl.ds(idx_row0, idx_rows_per_chunk)],
                            idx_v)
            copies = [
                pltpu.async_copy(
                    table_hbm.at[idx_v.at[j]],
                    rows_v.at[pl.ds(j * IDX_ROW, IDX_ROW)],
                    sem,
                )
                for j in range(n_streams)
            ]
            for c in copies:
                c.wait()

            def tok_body(t, c2):
                for kk in range(DIM // LANES):
                    s = pl.ds(kk * LANES, LANES)
                    out_v[t, s] = rows_v[2 * t, s] + rows_v[2 * t + 1, s]
                return c2

            lax.fori_loop(0, T_CHUNK, tok_body, 0, unroll=2)
            pltpu.sync_copy(out_v, out_hbm.at[pl.ds(base, T_CHUNK)])
            return carry

        lax.fori_loop(0, n_chunks, chunk_body, 0)

    return k


def kernel(batch_pos_list, table):
    B, L, P = batch_pos_list.shape
    assert P == 2
    V, D = table.shape
    assert D == DIM
    N = B * L
    idx2d = batch_pos_list.reshape(N * P // IDX_ROW, IDX_ROW)
    k = _make_kernel(N, V)
    out = k(idx2d, table)
    return out.reshape(B, L, D)


# speedup vs baseline: 3.7188x; 1.0028x over previous
import functools

import jax
import jax.numpy as jnp
from jax import lax
from jax.experimental import pallas as pl
from jax.experimental.pallas import tpu as pltpu
from jax.experimental.pallas import tpu_sc as plsc

DIM = 64
LANES = 16
IDX_ROW = 128
T_CHUNK = 512


def _make_kernel(num_tokens, vocab):
    info = plsc.get_sparse_core_info()
    num_workers = info.num_cores * info.num_subcores
    per_w = num_tokens // num_workers
    assert per_w * num_workers == num_tokens
    assert per_w % T_CHUNK == 0
    n_chunks = per_w // T_CHUNK
    n_streams = (2 * T_CHUNK) // IDX_ROW
    idx_rows_per_chunk = (2 * T_CHUNK) // IDX_ROW

    mesh = plsc.VectorSubcoreMesh(core_axis_name="c", subcore_axis_name="s")

    @functools.partial(
        pl.kernel,
        mesh=mesh,
        out_type=jax.ShapeDtypeStruct((num_tokens // 2, 2 * DIM), jnp.float32),
        scratch_types=[
            pltpu.VMEM((idx_rows_per_chunk, IDX_ROW), jnp.int32),
            pltpu.VMEM((2 * T_CHUNK, DIM), jnp.float32),
            pltpu.VMEM((T_CHUNK // 2, 2 * DIM), jnp.float32),
            pltpu.SemaphoreType.DMA,
        ],
        compiler_params=pltpu.CompilerParams(use_tc_tiling_on_sc=False),
    )
    def k(idx_hbm, table_hbm, out_hbm, idx_v, rows_v, out_v, sem):
        wid = lax.axis_index("s") * info.num_cores + lax.axis_index("c")
        tok0 = wid * per_w

        def chunk_body(g, carry):
            base = pl.multiple_of(tok0 + g * T_CHUNK, T_CHUNK)
            idx_row0 = pl.multiple_of((base * 2) // IDX_ROW, 8)
            pltpu.sync_copy(idx_hbm.at[pl.ds(idx_row0, idx_rows_per_chunk)],
                            idx_v)
            copies = [
                pltpu.async_copy(
                    table_hbm.at[idx_v.at[j]],
                    rows_v.at[pl.ds(j * IDX_ROW, IDX_ROW)],
                    sem,
                )
                for j in range(n_streams)
            ]
            for c in copies:
                c.wait()

            def pair_body(u, c2):
                for half in range(2):
                    for kk in range(DIM // LANES):
                        s_in = pl.ds(kk * LANES, LANES)
                        s_out = pl.ds(half * DIM + kk * LANES, LANES)
                        r0 = 4 * u + 2 * half
                        out_v[u, s_out] = rows_v[r0, s_in] + rows_v[r0 + 1, s_in]
                return c2

            lax.fori_loop(0, T_CHUNK // 2, pair_body, 0, unroll=2)
            pltpu.sync_copy(out_v, out_hbm.at[pl.ds(base // 2, T_CHUNK // 2)])
            return carry

        lax.fori_loop(0, n_chunks, chunk_body, 0)

    return k


def kernel(batch_pos_list, table):
    B, L, P = batch_pos_list.shape
    assert P == 2
    V, D = table.shape
    assert D == DIM
    N = B * L
    idx2d = batch_pos_list.reshape(N * P // IDX_ROW, IDX_ROW)
    k = _make_kernel(N, V)
    out = k(idx2d, table)
    return out.reshape(B, L, D)
